# Initial kernel scaffold; baseline (speedup 1.0000x reference)
import jax
import jax.numpy as jnp
from jax.experimental import pallas as pl


def kernel(table, input_ids, attention_mask):
    raise NotImplementedError("write your pallas kernel here")


# speedup vs baseline: 1.4629x; 1.4629x over previous
import functools

import jax
import jax.numpy as jnp
from jax import lax
from jax.experimental import pallas as pl
from jax.experimental.pallas import tpu as pltpu
from jax.experimental.pallas import tpu_sc as plsc

_NC = 2
_NS = 16
_NW = _NC * _NS
_LANES = 16

_B, _S, _H = 4, 2048, 1536
_NTOK = _B * _S
_PER_W = _NTOK // _NW
_CHUNK = 32
_NCHUNK = _PER_W // _CHUNK
_SCHUNKS = _S // _LANES


def _make_sc_call():
    mesh = plsc.VectorSubcoreMesh(core_axis_name="c", subcore_axis_name="s",
                                  num_cores=_NC, num_subcores=_NS)
    return pl.kernel(
        _sc_body_wrapper,
        out_type=(
            jax.ShapeDtypeStruct((_NTOK, _H), jnp.float32),
            jax.ShapeDtypeStruct((_B, _H), jnp.float32),
        ),
        mesh=mesh,
        scratch_types=[
            pltpu.VMEM((_PER_W,), jnp.int32),
            pltpu.VMEM((_CHUNK, _H), jnp.float32),
            pltpu.VMEM((_CHUNK, _H), jnp.float32),
            pltpu.VMEM((_S,), jnp.int32),
            pltpu.VMEM((_LANES,), jnp.int32),
            pltpu.SemaphoreType.DMA,
            pltpu.SemaphoreType.DMA,
        ],
    )


def _sc_body_wrapper(table_hbm, ids_hbm, ids2_hbm, mask_hbm,
                     out_hbm, last_hbm,
                     idx_v, buf0, buf1, row_v, lastidx_v, sem0, sem1):
    wid = lax.axis_index("s") * _NC + lax.axis_index("c")
    base = pl.multiple_of(wid * _PER_W, _PER_W)

    pltpu.sync_copy(ids_hbm.at[pl.ds(base, _PER_W)], idx_v)

    bufs = (buf0, buf1)
    sems = (sem0, sem1)
    handles = [None, None]
    for c in range(_NCHUNK):
        handles[c % 2] = pltpu.async_copy(
            table_hbm.at[idx_v.at[pl.ds(c * _CHUNK, _CHUNK)]],
            bufs[c % 2], sems[c % 2])
        if c > 0:
            handles[(c - 1) % 2].wait()
            pltpu.sync_copy(
                bufs[(c - 1) % 2],
                out_hbm.at[pl.ds(base + (c - 1) * _CHUNK, _CHUNK)])
    handles[(_NCHUNK - 1) % 2].wait()
    pltpu.sync_copy(
        bufs[(_NCHUNK - 1) % 2],
        out_hbm.at[pl.ds(base + (_NCHUNK - 1) * _CHUNK, _CHUNK)])

    @pl.when(wid < _B)
    def _last():
        pltpu.sync_copy(mask_hbm.at[wid], row_v)

        def _sum_body(i, acc):
            off = pl.multiple_of(i * _LANES, _LANES)
            return acc + row_v[pl.ds(off, _LANES)]

        acc = lax.fori_loop(0, _SCHUNKS, _sum_body,
                            jnp.zeros((_LANES,), jnp.int32))
        total = acc[0]
        for i in range(1, _LANES):
            total = total + acc[i]
        pos = total - 1

        pltpu.sync_copy(ids2_hbm.at[wid], row_v)

        def _pick_body(i, best):
            off = pl.multiple_of(i * _LANES, _LANES)
            v = row_v[pl.ds(off, _LANES)]
            lane_pos = lax.iota(jnp.int32, _LANES) + off
            return jnp.maximum(best, jnp.where(lane_pos == pos, v, -1))

        best = lax.fori_loop(0, _SCHUNKS, _pick_body,
                             jnp.full((_LANES,), -1, jnp.int32))
        tid = best[0]
        for i in range(1, _LANES):
            tid = jnp.maximum(tid, best[i])

        lastidx_v[...] = jnp.full((_LANES,), tid, jnp.int32)
        pltpu.async_copy(table_hbm.at[lastidx_v],
                         buf0.at[pl.ds(0, _LANES)], sem0).wait()
        pltpu.sync_copy(buf0.at[0], last_hbm.at[wid])


@jax.jit
def _run(table, ids_flat, ids_2d, mask_2d):
    out_flat, last = _make_sc_call()(table, ids_flat, ids_2d, mask_2d)
    return out_flat, last


def kernel(table, input_ids, attention_mask):
    ids_2d = input_ids.astype(jnp.int32)
    ids_flat = ids_2d.reshape(-1)
    mask_2d = attention_mask.astype(jnp.int32)
    out_flat, last = _run(table, ids_flat, ids_2d, mask_2d)
    return out_flat.reshape(_B, _S, _H), last


# speedup vs baseline: 1.4632x; 1.0002x over previous
import functools

import jax
import jax.numpy as jnp
from jax import lax
from jax.experimental import pallas as pl
from jax.experimental.pallas import tpu as pltpu
from jax.experimental.pallas import tpu_sc as plsc

_NC = 2
_NS = 16
_NW = _NC * _NS
_LANES = 16

_B, _S, _H = 4, 2048, 1536
_NTOK = _B * _S
_PER_W = _NTOK // _NW
_CHUNK = 32
_NCHUNK = _PER_W // _CHUNK
_SCHUNKS = _S // _LANES


def _make_sc_call():
    mesh = plsc.VectorSubcoreMesh(core_axis_name="c", subcore_axis_name="s",
                                  num_cores=_NC, num_subcores=_NS)
    return pl.kernel(
        _sc_body_wrapper,
        out_type=(
            jax.ShapeDtypeStruct((_NTOK, _H), jnp.float32),
            jax.ShapeDtypeStruct((_B, _H), jnp.float32),
        ),
        mesh=mesh,
        scratch_types=[
            pltpu.VMEM((_PER_W,), jnp.int32),
            pltpu.VMEM((_CHUNK, _H), jnp.float32),
            pltpu.VMEM((_CHUNK, _H), jnp.float32),
            pltpu.VMEM((_S,), jnp.int32),
            pltpu.VMEM((_LANES,), jnp.int32),
            pltpu.SemaphoreType.DMA,
            pltpu.SemaphoreType.DMA,
            pltpu.SemaphoreType.DMA,
            pltpu.SemaphoreType.DMA,
        ],
    )


def _sc_body_wrapper(table_hbm, ids_hbm, ids2_hbm, mask_hbm,
                     out_hbm, last_hbm,
                     idx_v, buf0, buf1, row_v, lastidx_v,
                     sem0, sem1, osem0, osem1):
    wid = lax.axis_index("s") * _NC + lax.axis_index("c")
    base = pl.multiple_of(wid * _PER_W, _PER_W)

    pltpu.sync_copy(ids_hbm.at[pl.ds(base, _PER_W)], idx_v)

    bufs = (buf0, buf1)
    gsems = (sem0, sem1)
    osems = (osem0, osem1)
    gh = [None, None]
    oh = [None, None]
    for c in range(_NCHUNK):
        if oh[c % 2] is not None:
            oh[c % 2].wait()
        gh[c % 2] = pltpu.async_copy(
            table_hbm.at[idx_v.at[pl.ds(c * _CHUNK, _CHUNK)]],
            bufs[c % 2], gsems[c % 2])
        if c > 0:
            gh[(c - 1) % 2].wait()
            oh[(c - 1) % 2] = pltpu.async_copy(
                bufs[(c - 1) % 2],
                out_hbm.at[pl.ds(base + (c - 1) * _CHUNK, _CHUNK)],
                osems[(c - 1) % 2])
    last_c = _NCHUNK - 1
    gh[last_c % 2].wait()
    oh[last_c % 2] = pltpu.async_copy(
        bufs[last_c % 2],
        out_hbm.at[pl.ds(base + last_c * _CHUNK, _CHUNK)],
        osems[last_c % 2])
    oh[0].wait()
    oh[1].wait()

    @pl.when(wid < _B)
    def _last():
        pltpu.sync_copy(mask_hbm.at[wid], row_v)

        def _sum_body(i, acc):
            off = pl.multiple_of(i * _LANES, _LANES)
            return acc + row_v[pl.ds(off, _LANES)]

        acc = lax.fori_loop(0, _SCHUNKS, _sum_body,
                            jnp.zeros((_LANES,), jnp.int32))
        total = acc[0]
        for i in range(1, _LANES):
            total = total + acc[i]
        pos = total - 1

        pltpu.sync_copy(ids2_hbm.at[wid], row_v)

        def _pick_body(i, best):
            off = pl.multiple_of(i * _LANES, _LANES)
            v = row_v[pl.ds(off, _LANES)]
            lane_pos = lax.iota(jnp.int32, _LANES) + off
            return jnp.maximum(best, jnp.where(lane_pos == pos, v, -1))

        best = lax.fori_loop(0, _SCHUNKS, _pick_body,
                             jnp.full((_LANES,), -1, jnp.int32))
        tid = best[0]
        for i in range(1, _LANES):
            tid = jnp.maximum(tid, best[i])

        lastidx_v[...] = jnp.full((_LANES,), tid, jnp.int32)
        pltpu.async_copy(table_hbm.at[lastidx_v],
                         buf0.at[pl.ds(0, _LANES)], sem0).wait()
        pltpu.sync_copy(buf0.at[0], last_hbm.at[wid])


@jax.jit
def _run(table, ids_flat, ids_2d, mask_2d):
    out_flat, last = _make_sc_call()(table, ids_flat, ids_2d, mask_2d)
    return out_flat, last


def kernel(table, input_ids, attention_mask):
    ids_2d = input_ids.astype(jnp.int32)
    ids_flat = ids_2d.reshape(-1)
    mask_2d = attention_mask.astype(jnp.int32)
    out_flat, last = _run(table, ids_flat, ids_2d, mask_2d)
    return out_flat.reshape(_B, _S, _H), last


# speedup vs baseline: 1.5590x; 1.0655x over previous
import functools

import jax
import jax.numpy as jnp
from jax import lax
from jax.experimental import pallas as pl
from jax.experimental.pallas import tpu as pltpu
from jax.experimental.pallas import tpu_sc as plsc

_NC = 2
_NS = 16
_NW = _NC * _NS
_LANES = 16

_B, _S, _H = 4, 2048, 1536
_NTOK = _B * _S
_PER_W = _NTOK // _NW
_CHUNK = 32
_NCHUNK = _PER_W // _CHUNK
_SCHUNKS = _S // _LANES


def _make_sc_call():
    mesh = plsc.VectorSubcoreMesh(core_axis_name="c", subcore_axis_name="s",
                                  num_cores=_NC, num_subcores=_NS)
    return pl.kernel(
        _sc_body_wrapper,
        out_type=(
            jax.ShapeDtypeStruct((_NTOK, _H), jnp.float32),
            jax.ShapeDtypeStruct((_B, _H), jnp.float32),
        ),
        mesh=mesh,
        scratch_types=[
            pltpu.VMEM((_PER_W,), jnp.int32),
            pltpu.VMEM((_CHUNK, _H), jnp.float32),
            pltpu.VMEM((_CHUNK, _H), jnp.float32),
            pltpu.VMEM((_S,), jnp.int32),
            pltpu.VMEM((_LANES,), jnp.int32),
            pltpu.VMEM((_LANES, _H), jnp.float32),
            pltpu.SMEM((2,), jnp.int32),
            pltpu.SemaphoreType.DMA,
            pltpu.SemaphoreType.DMA,
            pltpu.SemaphoreType.DMA,
            pltpu.SemaphoreType.DMA,
            pltpu.SemaphoreType.DMA,
            pltpu.SemaphoreType.DMA,
        ],
    )


def _sc_body_wrapper(table_hbm, ids_hbm, ids2_hbm, mask_hbm,
                     out_hbm, last_hbm,
                     idx_v, buf0, buf1, row_v, lastidx_v, lastbuf, pos_smem,
                     sem0, sem1, osem0, osem1, msem, lsem):
    wid = lax.axis_index("s") * _NC + lax.axis_index("c")
    base = pl.multiple_of(wid * _PER_W, _PER_W)
    is_last_worker = wid < _B

    pltpu.sync_copy(ids_hbm.at[pl.ds(base, _PER_W)], idx_v)

    @pl.when(is_last_worker)
    def _start_mask():
        pltpu.async_copy(mask_hbm.at[wid], row_v, msem)

    bufs = (buf0, buf1)
    gsems = (sem0, sem1)
    osems = (osem0, osem1)
    gh = [None, None]
    oh = [None, None]
    for c in range(_NCHUNK):
        if oh[c % 2] is not None:
            oh[c % 2].wait()
        gh[c % 2] = pltpu.async_copy(
            table_hbm.at[idx_v.at[pl.ds(c * _CHUNK, _CHUNK)]],
            bufs[c % 2], gsems[c % 2])

        if c == 0:
            @pl.when(is_last_worker)
            def _mask_sum():
                pltpu.make_async_copy(mask_hbm.at[wid], row_v, msem).wait()

                def _sum_body(i, acc):
                    off = pl.multiple_of(i * _LANES, _LANES)
                    return acc + row_v[pl.ds(off, _LANES)]

                acc = lax.fori_loop(0, _SCHUNKS, _sum_body,
                                    jnp.zeros((_LANES,), jnp.int32))
                total = acc[0]
                for i in range(1, _LANES):
                    total = total + acc[i]
                pos_smem[0] = total - 1
                pltpu.async_copy(ids2_hbm.at[wid], row_v, msem)
        elif c == 1:
            @pl.when(is_last_worker)
            def _pick_tid():
                pltpu.make_async_copy(ids2_hbm.at[wid], row_v, msem).wait()
                pos = pos_smem[0]

                def _pick_body(i, best):
                    off = pl.multiple_of(i * _LANES, _LANES)
                    v = row_v[pl.ds(off, _LANES)]
                    lane_pos = lax.iota(jnp.int32, _LANES) + off
                    return jnp.maximum(best,
                                       jnp.where(lane_pos == pos, v, -1))

                best = lax.fori_loop(0, _SCHUNKS, _pick_body,
                                     jnp.full((_LANES,), -1, jnp.int32))
                tid = best[0]
                for i in range(1, _LANES):
                    tid = jnp.maximum(tid, best[i])
                lastidx_v[...] = jnp.full((_LANES,), tid, jnp.int32)
                pltpu.async_copy(table_hbm.at[lastidx_v], lastbuf, lsem)
        elif c == 2:
            @pl.when(is_last_worker)
            def _emit_last():
                pltpu.make_async_copy(table_hbm.at[lastidx_v], lastbuf,
                                      lsem).wait()
                pltpu.sync_copy(lastbuf.at[0], last_hbm.at[wid])

        if c > 0:
            gh[(c - 1) % 2].wait()
            oh[(c - 1) % 2] = pltpu.async_copy(
                bufs[(c - 1) % 2],
                out_hbm.at[pl.ds(base + (c - 1) * _CHUNK, _CHUNK)],
                osems[(c - 1) % 2])
    last_c = _NCHUNK - 1
    gh[last_c % 2].wait()
    oh[last_c % 2] = pltpu.async_copy(
        bufs[last_c % 2],
        out_hbm.at[pl.ds(base + last_c * _CHUNK, _CHUNK)],
        osems[last_c % 2])
    oh[0].wait()
    oh[1].wait()


@jax.jit
def _run(table, ids_flat, ids_2d, mask_2d):
    out_flat, last = _make_sc_call()(table, ids_flat, ids_2d, mask_2d)
    return out_flat, last


def kernel(table, input_ids, attention_mask):
    ids_2d = input_ids.astype(jnp.int32)
    ids_flat = ids_2d.reshape(-1)
    mask_2d = attention_mask.astype(jnp.int32)
    out_flat, last = _run(table, ids_flat, ids_2d, mask_2d)
    return out_flat.reshape(_B, _S, _H), last


# speedup vs baseline: 1.6097x; 1.0325x over previous
import jax
import jax.numpy as jnp
from jax import lax
from jax.experimental import pallas as pl
from jax.experimental.pallas import tpu as pltpu
from jax.experimental.pallas import tpu_sc as plsc

_NC = 2
_NS = 16
_NW = _NC * _NS
_LANES = 16

_B, _S, _H = 4, 2048, 1536
_NTOK = _B * _S
_PER_W = _NTOK // _NW
_CHUNKS = (40, 40, 40, 40, 40, 40, 16)
_MAXCHUNK = max(_CHUNKS)
_OFFS = tuple(sum(_CHUNKS[:i]) for i in range(len(_CHUNKS)))
_NCHUNK = len(_CHUNKS)
_NBUF = 2
_SCHUNKS = _S // _LANES


def _make_sc_call():
    mesh = plsc.VectorSubcoreMesh(core_axis_name="c", subcore_axis_name="s",
                                  num_cores=_NC, num_subcores=_NS)
    scratch = [pltpu.VMEM((_PER_W,), jnp.int32)]
    scratch += [pltpu.VMEM((_MAXCHUNK, _H), jnp.float32)
                for _ in range(_NBUF)]
    scratch += [
        pltpu.VMEM((_S,), jnp.int32),
        pltpu.VMEM((_LANES,), jnp.int32),
        pltpu.VMEM((1, _H), jnp.float32),
        pltpu.SMEM((2,), jnp.int32),
    ]
    scratch += [pltpu.SemaphoreType.DMA] * (2 * _NBUF + 2)
    return pl.kernel(
        _sc_body,
        out_type=(
            jax.ShapeDtypeStruct((_NTOK, _H), jnp.float32),
            jax.ShapeDtypeStruct((_B, _H), jnp.float32),
        ),
        mesh=mesh,
        scratch_types=scratch,
    )


def _sc_body(table_hbm, ids_hbm, ids2_hbm, mask_hbm, out_hbm, last_hbm,
             idx_v, *scratch):
    bufs = scratch[:_NBUF]
    row_v, lastidx_v, lastbuf, pos_smem = scratch[_NBUF:_NBUF + 4]
    gsems = scratch[_NBUF + 4:2 * _NBUF + 4]
    osems = scratch[2 * _NBUF + 4:3 * _NBUF + 4]
    msem, lsem = scratch[3 * _NBUF + 4:]

    wid = lax.axis_index("s") * _NC + lax.axis_index("c")
    base = pl.multiple_of(wid * _PER_W, _PER_W)
    is_last_worker = wid < _B

    pltpu.sync_copy(ids_hbm.at[pl.ds(base, _PER_W)], idx_v)

    @pl.when(is_last_worker)
    def _start_mask():
        pltpu.async_copy(mask_hbm.at[wid], row_v, msem)

    gh = [None] * _NBUF
    oh = [None] * _NBUF
    for c in range(_NCHUNK):
        s = c % _NBUF
        if oh[s] is not None:
            oh[s].wait()
        gh[s] = pltpu.async_copy(
            table_hbm.at[idx_v.at[pl.ds(_OFFS[c], _CHUNKS[c])]],
            bufs[s].at[pl.ds(0, _CHUNKS[c])], gsems[s])

        if c == 0:
            @pl.when(is_last_worker)
            def _mask_sum():
                pltpu.make_async_copy(mask_hbm.at[wid], row_v, msem).wait()

                def _sum_body(i, acc):
                    off = pl.multiple_of(i * _LANES, _LANES)
                    return acc + row_v[pl.ds(off, _LANES)]

                acc = lax.fori_loop(0, _SCHUNKS, _sum_body,
                                    jnp.zeros((_LANES,), jnp.int32))
                total = acc[0]
                for i in range(1, _LANES):
                    total = total + acc[i]
                pos_smem[0] = total - 1
                pltpu.async_copy(ids2_hbm.at[wid], row_v, msem)
        elif c == 1:
            @pl.when(is_last_worker)
            def _pick_tid():
                pltpu.make_async_copy(ids2_hbm.at[wid], row_v, msem).wait()
                pos = pos_smem[0]

                def _pick_body(i, best):
                    off = pl.multiple_of(i * _LANES, _LANES)
                    v = row_v[pl.ds(off, _LANES)]
                    lane_pos = lax.iota(jnp.int32, _LANES) + off
                    return jnp.maximum(best,
                                       jnp.where(lane_pos == pos, v, -1))

                best = lax.fori_loop(0, _SCHUNKS, _pick_body,
                                     jnp.full((_LANES,), -1, jnp.int32))
                tid = best[0]
                for i in range(1, _LANES):
                    tid = jnp.maximum(tid, best[i])
                lastidx_v[...] = jnp.full((_LANES,), tid, jnp.int32)
                pltpu.async_copy(table_hbm.at[lastidx_v.at[pl.ds(0, 1)]],
                                 lastbuf, lsem)
        elif c == 2:
            @pl.when(is_last_worker)
            def _emit_last():
                pltpu.make_async_copy(table_hbm.at[lastidx_v.at[pl.ds(0, 1)]],
                                      lastbuf, lsem).wait()
                pltpu.sync_copy(lastbuf.at[0], last_hbm.at[wid])

        if c > 0:
            ps = (c - 1) % _NBUF
            gh[ps].wait()
            oh[ps] = pltpu.async_copy(
                bufs[ps].at[pl.ds(0, _CHUNKS[c - 1])],
                out_hbm.at[pl.ds(base + _OFFS[c - 1], _CHUNKS[c - 1])],
                osems[ps])
    last_c = _NCHUNK - 1
    s = last_c % _NBUF
    gh[s].wait()
    oh[s] = pltpu.async_copy(
        bufs[s].at[pl.ds(0, _CHUNKS[last_c])],
        out_hbm.at[pl.ds(base + _OFFS[last_c], _CHUNKS[last_c])], osems[s])
    for s in range(_NBUF):
        if oh[s] is not None:
            oh[s].wait()


@jax.jit
def _run(table, ids_flat, ids_2d, mask_2d):
    out_flat, last = _make_sc_call()(table, ids_flat, ids_2d, mask_2d)
    return out_flat, last


def kernel(table, input_ids, attention_mask):
    ids_2d = input_ids.astype(jnp.int32)
    ids_flat = ids_2d.reshape(-1)
    mask_2d = attention_mask.astype(jnp.int32)
    out_flat, last = _run(table, ids_flat, ids_2d, mask_2d)
    return out_flat.reshape(_B, _S, _H), last
